# baseline (speedup 1.0000x reference)
import functools

import jax
import jax.numpy as jnp
from jax import lax
from jax.experimental import pallas as pl
from jax.experimental.pallas import tpu as pltpu
from jax.experimental.pallas import tpu_sc as plsc

_BATCH = 4096
_SEQ = 200
_D = 128
_B = _BATCH * _SEQ
_NW = 32
_B_PER_W = _B // _NW
_CHUNK = 128
_N_CHUNKS = _B_PER_W // _CHUNK
_NBUF = 6
_K = 3


def _emb_body(idx_hbm, table_hbm, out_hbm, idx_v, rows, gsems, ssems):
    wid = lax.axis_index("s") * 2 + lax.axis_index("c")
    base = wid * _B_PER_W
    pltpu.sync_copy(idx_hbm.at[wid], idx_v)

    def gather(j, b):
        pltpu.async_copy(table_hbm.at[idx_v.at[j]], rows[b], gsems[b])

    def gwait(b):
        pltpu.make_async_copy(table_hbm.at[idx_v.at[0]], rows[b],
                              gsems[b]).wait()

    def store(j, b):
        pltpu.async_copy(rows[b],
                         out_hbm.at[pl.ds(base + j * _CHUNK, _CHUNK)],
                         ssems[b])

    def swait(b):
        pltpu.make_async_copy(rows[b],
                              out_hbm.at[pl.ds(base, _CHUNK)],
                              ssems[b]).wait()

    def step(j, b, do_swait, do_gather):
        nb = (b + _K) % _NBUF
        if do_swait:
            swait(nb)
        if do_gather:
            gather(j + _K, nb)
        gwait(b)
        store(j, b)

    for j in range(_K):
        gather(j, j)
    for j in range(_NBUF):
        step(j, j, j + _K >= _NBUF, True)

    def body(t, _):
        for b in range(_NBUF):
            step(t * _NBUF + b, b, True, True)
        return 0

    _ep_start = ((_N_CHUNKS - _K) // _NBUF) * _NBUF
    lax.fori_loop(1, _ep_start // _NBUF, body, 0)

    for j in range(_ep_start, _N_CHUNKS):
        do_g = j + _K < _N_CHUNKS
        step(j, j % _NBUF, do_g, do_g)
    for b in range(_NBUF):
        swait(b)


_emb_kernel = functools.partial(
    pl.kernel,
    out_type=jax.ShapeDtypeStruct((_B, _D), jnp.float32),
    mesh=plsc.VectorSubcoreMesh(core_axis_name="c", subcore_axis_name="s"),
    scratch_types=[
        pltpu.VMEM((_N_CHUNKS, _CHUNK), jnp.int32),
        [pltpu.VMEM((_CHUNK, _D), jnp.float32)] * _NBUF,
        [pltpu.SemaphoreType.DMA] * _NBUF,
        [pltpu.SemaphoreType.DMA] * _NBUF,
    ],
)(_emb_body)


def kernel(actions, emb_weight):
    idx = actions.reshape(_NW, _N_CHUNKS, _CHUNK).astype(jnp.int32)
    out = _emb_kernel(idx, emb_weight)
    return out.reshape(_BATCH, _SEQ, _D)


# speedup vs baseline: 2.3657x; 2.3657x over previous
import functools

import jax
import jax.numpy as jnp
from jax import lax
from jax.experimental import pallas as pl
from jax.experimental.pallas import tpu as pltpu
from jax.experimental.pallas import tpu_sc as plsc

_BATCH = 4096
_SEQ = 200
_D = 128
_B = _BATCH * _SEQ
_NW = 32
_B_PER_W = _B // _NW
_CHUNK = 128
_N_CHUNKS = _B_PER_W // _CHUNK
_NBUF = 5
_K = 3
_V = 1000


def _emb_body(idx_hbm, table_hbm, out_hbm, idx_v, rows, table_sp, gsems,
              ssems):
    sid = lax.axis_index("s")
    wid = sid * 2 + lax.axis_index("c")
    base = wid * _B_PER_W
    @pl.when(sid == 0)
    def _():
        pltpu.sync_copy(table_hbm, table_sp)
    pltpu.sync_copy(idx_hbm.at[wid], idx_v)
    plsc.subcore_barrier()

    def gather(j, b):
        pltpu.async_copy(table_sp.at[idx_v.at[j]], rows[b], gsems[b])

    def gwait(b):
        pltpu.make_async_copy(table_sp.at[idx_v.at[0]], rows[b],
                              gsems[b]).wait()

    def store(j, b):
        pltpu.async_copy(rows[b],
                         out_hbm.at[pl.ds(base + j * _CHUNK, _CHUNK)],
                         ssems[b])

    def swait(b):
        pltpu.make_async_copy(rows[b],
                              out_hbm.at[pl.ds(base, _CHUNK)],
                              ssems[b]).wait()

    def step(j, b, do_swait, do_gather):
        nb = (b + _K) % _NBUF
        if do_swait:
            swait(nb)
        if do_gather:
            gather(j + _K, nb)
        gwait(b)
        store(j, b)

    for j in range(_K):
        gather(j, j)
    for j in range(_NBUF):
        step(j, j, j + _K >= _NBUF, True)

    def body(t, _):
        for b in range(_NBUF):
            step(t * _NBUF + b, b, True, True)
        return 0

    _ep_start = ((_N_CHUNKS - _K) // _NBUF) * _NBUF
    lax.fori_loop(1, _ep_start // _NBUF, body, 0)

    for j in range(_ep_start, _N_CHUNKS):
        do_g = j + _K < _N_CHUNKS
        step(j, j % _NBUF, do_g, do_g)
    for b in range(_NBUF):
        swait(b)


_emb_kernel = functools.partial(
    pl.kernel,
    out_type=jax.ShapeDtypeStruct((_B, _D), jnp.float32),
    mesh=plsc.VectorSubcoreMesh(core_axis_name="c", subcore_axis_name="s"),
    scratch_types=[
        pltpu.VMEM((_N_CHUNKS, _CHUNK), jnp.int32),
        [pltpu.VMEM((_CHUNK, _D), jnp.float32)] * _NBUF,
        pltpu.VMEM_SHARED((_V, _D), jnp.float32),
        [pltpu.SemaphoreType.DMA] * _NBUF,
        [pltpu.SemaphoreType.DMA] * _NBUF,
    ],
)(_emb_body)


def kernel(actions, emb_weight):
    idx = actions.reshape(_NW, _N_CHUNKS, _CHUNK).astype(jnp.int32)
    out = _emb_kernel(idx, emb_weight)
    return out.reshape(_BATCH, _SEQ, _D)
